# speedup vs baseline: 3.1880x; 3.1880x over previous
import functools

import jax
import jax.numpy as jnp
from jax import lax
from jax.experimental import pallas as pl
from jax.experimental.pallas import tpu as pltpu
from jax.experimental.pallas import tpu_sc as plsc

VOCAB = 30522
DIM = 128


@functools.lru_cache(maxsize=None)
def _make_gather(B: int, D: int):
    info = plsc.get_sparse_core_info()
    NC, NS = info.num_cores, info.num_subcores
    NW = NC * NS
    assert B % NW == 0
    b_per_w = B // NW
    chunk = 400
    nchunks = b_per_w // chunk
    assert b_per_w % chunk == 0 and chunk % 8 == 0

    mesh = plsc.VectorSubcoreMesh(core_axis_name="c", subcore_axis_name="s")

    @functools.partial(
        pl.kernel,
        mesh=mesh,
        out_type=jax.ShapeDtypeStruct((B, D), jnp.float32),
        scratch_types=[
            pltpu.VMEM((b_per_w,), jnp.int32),
            pltpu.VMEM((chunk, D), jnp.float32),
            pltpu.SemaphoreType.DMA,
        ],
    )
    def k(idx_hbm, table_hbm, out_hbm, idx_v, rows_v, sem):
        wid = lax.axis_index("s") * NC + lax.axis_index("c")
        base = wid * b_per_w
        pltpu.sync_copy(idx_hbm.at[pl.ds(base, b_per_w)], idx_v)

        def body(g, carry):
            off = pl.multiple_of(g * chunk, 8)
            pltpu.async_copy(
                table_hbm.at[idx_v.at[pl.ds(off, chunk)]], rows_v, sem
            ).wait()
            pltpu.sync_copy(rows_v, out_hbm.at[pl.ds(base + off, chunk)])
            return carry

        lax.fori_loop(0, nchunks, body, 0)

    return k


def kernel(x, table):
    B = x.shape[0] * x.shape[1]
    idx = x.reshape(B).astype(jnp.int32)
    out = _make_gather(B, DIM)(idx, table)
    return out.reshape(x.shape[0], x.shape[1], DIM)


# speedup vs baseline: 3.2528x; 1.0203x over previous
import functools

import jax
import jax.numpy as jnp
from jax import lax
from jax.experimental import pallas as pl
from jax.experimental.pallas import tpu as pltpu
from jax.experimental.pallas import tpu_sc as plsc

VOCAB = 30522
DIM = 128


@functools.lru_cache(maxsize=None)
def _make_gather(B: int, D: int):
    info = plsc.get_sparse_core_info()
    NC, NS = info.num_cores, info.num_subcores
    NW = NC * NS
    assert B % NW == 0
    b_per_w = B // NW
    chunk = 400
    nchunks = b_per_w // chunk
    assert b_per_w % chunk == 0 and chunk % 8 == 0

    mesh = plsc.VectorSubcoreMesh(core_axis_name="c", subcore_axis_name="s")

    @functools.partial(
        pl.kernel,
        mesh=mesh,
        out_type=jax.ShapeDtypeStruct((B, D), jnp.float32),
        scratch_types=[
            pltpu.VMEM((b_per_w,), jnp.int32),
            pltpu.VMEM((2, chunk, D), jnp.float32),
            pltpu.SemaphoreType.DMA,
            pltpu.SemaphoreType.DMA,
            pltpu.SemaphoreType.DMA,
            pltpu.SemaphoreType.DMA,
        ],
    )
    def k(idx_hbm, table_hbm, out_hbm, idx_v, rows_v, g0, g1, w0, w1):
        wid = lax.axis_index("s") * NC + lax.axis_index("c")
        base = wid * b_per_w
        gsem = (g0, g1)
        wsem = (w0, w1)
        pltpu.sync_copy(idx_hbm.at[pl.ds(base, b_per_w)], idx_v)

        gathers = [None, None]
        writes = [None, None]
        gathers[0] = pltpu.async_copy(
            table_hbm.at[idx_v.at[pl.ds(0, chunk)]], rows_v.at[0], gsem[0]
        )
        for g in range(nchunks):
            b = g % 2
            gathers[b].wait()
            if writes[1 - b] is not None:
                writes[1 - b].wait()
            if g + 1 < nchunks:
                gathers[1 - b] = pltpu.async_copy(
                    table_hbm.at[idx_v.at[pl.ds((g + 1) * chunk, chunk)]],
                    rows_v.at[1 - b],
                    gsem[1 - b],
                )
            writes[b] = pltpu.async_copy(
                rows_v.at[b], out_hbm.at[pl.ds(base + g * chunk, chunk)], wsem[b]
            )
        writes[(nchunks - 1) % 2].wait()

    return k


def kernel(x, table):
    B = x.shape[0] * x.shape[1]
    idx = x.reshape(B).astype(jnp.int32)
    out = _make_gather(B, DIM)(idx, table)
    return out.reshape(x.shape[0], x.shape[1], DIM)


# speedup vs baseline: 10.1298x; 3.1141x over previous
import functools

import jax
import jax.numpy as jnp
from jax import lax
from jax.experimental import pallas as pl
from jax.experimental.pallas import tpu as pltpu
from jax.experimental.pallas import tpu_sc as plsc

VOCAB = 30522
DIM = 128


@functools.lru_cache(maxsize=None)
def _make_gather(B: int, D: int):
    info = plsc.get_sparse_core_info()
    NC, NS = info.num_cores, info.num_subcores
    NW = NC * NS
    assert B % NW == 0
    b_per_w = B // NW
    chunk = 400
    nchunks = b_per_w // chunk
    assert b_per_w % chunk == 0 and chunk % 8 == 0

    mesh = plsc.VectorSubcoreMesh(core_axis_name="c", subcore_axis_name="s")

    @functools.partial(
        pl.kernel,
        mesh=mesh,
        out_type=jax.ShapeDtypeStruct((B, D), jnp.float32),
        scratch_types=[
            pltpu.VMEM((b_per_w,), jnp.int32),
            pltpu.VMEM((2, chunk, D), jnp.float32),
            pltpu.SemaphoreType.DMA,
            pltpu.SemaphoreType.DMA,
            pltpu.SemaphoreType.DMA,
            pltpu.SemaphoreType.DMA,
        ],
    )
    def k(idx_hbm, table_hbm, out_hbm, idx_v, rows_v, g0, g1, w0, w1):
        wid = lax.axis_index("s") * NC + lax.axis_index("c")
        base = wid * b_per_w
        gsem = (g0, g1)
        wsem = (w0, w1)
        pltpu.sync_copy(idx_hbm.at[pl.ds(base, b_per_w)], idx_v)

        gathers = [None, None]
        writes = [None, None]
        gathers[0] = pltpu.async_copy(
            table_hbm.at[idx_v.at[pl.ds(0, chunk)]], rows_v.at[0], gsem[0]
        )
        for g in range(nchunks):
            b = g % 2
            gathers[b].wait()
            if writes[1 - b] is not None:
                writes[1 - b].wait()
            if g + 1 < nchunks:
                gathers[1 - b] = pltpu.async_copy(
                    table_hbm.at[idx_v.at[pl.ds((g + 1) * chunk, chunk)]],
                    rows_v.at[1 - b],
                    gsem[1 - b],
                )
            writes[b] = pltpu.async_copy(
                rows_v.at[b], out_hbm.at[pl.ds(base + g * chunk, chunk)], wsem[b]
            )
        writes[(nchunks - 1) % 2].wait()

    return k


def kernel(x, table):
    nb, ns = x.shape
    B = nb * ns
    idx = x.T.reshape(B).astype(jnp.int32)
    out = _make_gather(B, DIM)(idx, table)
    return out.reshape(ns, nb, DIM).transpose(1, 0, 2)
